# Initial kernel scaffold; baseline (speedup 1.0000x reference)
import jax
import jax.numpy as jnp
from jax.experimental import pallas as pl


def kernel(dp, f_cf, patch_mask):
    raise NotImplementedError("write your pallas kernel here")


# speedup vs baseline: 5.3147x; 5.3147x over previous
import jax
import jax.numpy as jnp
from jax.experimental import pallas as pl
from jax.experimental.pallas import tpu as pltpu

TAU = 0.07
MIN_PIXELS = 3
LAMBDA_CF = 0.5
NUM_INST = 8
NSEG = 16

_ROWS = 4000


def _norm_rows(x):
    n = jnp.sqrt(jnp.sum(x * x, axis=1, keepdims=True))
    return x / jnp.maximum(n, 1e-12)


def _pass1(dp_ref, cf_ref, lab_ref, segdp_ref, segcf_ref, cnt_ref):
    step = pl.program_id(0)
    xn = _norm_rows(dp_ref[...])
    yn = _norm_rows(cf_ref[...])
    lab = lab_ref[0, 0, :]
    r = xn.shape[0]
    onehot = (lab[:, None] == jax.lax.broadcasted_iota(jnp.int32, (r, NSEG), 1)
              ).astype(jnp.float32)
    sdp = jax.lax.dot_general(onehot, xn, (((0,), (0,)), ((), ())),
                              preferred_element_type=jnp.float32)
    scf = jax.lax.dot_general(onehot, yn, (((0,), (0,)), ((), ())),
                              preferred_element_type=jnp.float32)
    cnt = jnp.sum(onehot, axis=0)

    @pl.when(step == 0)
    def _():
        segdp_ref[...] = jnp.zeros_like(segdp_ref)
        segcf_ref[...] = jnp.zeros_like(segcf_ref)
        cnt_ref[...] = jnp.zeros_like(cnt_ref)

    segdp_ref[...] += sdp
    segcf_ref[...] += scf
    cnt_ref[...] += jnp.broadcast_to(cnt[:, None], cnt_ref.shape)


def _pass2(dp_ref, cf_ref, lab_ref, segdp_ref, segcf_ref, cnt_ref,
           tsum_ref, csum_ref):
    step = pl.program_id(0)
    counts = cnt_ref[:, 0:1]
    safe = jnp.maximum(counts, 1.0)
    mu_dp = _norm_rows(segdp_ref[...] / safe)
    mu_cf = _norm_rows(segcf_ref[...] / safe)

    xn = _norm_rows(dp_ref[...])
    yn = _norm_rows(cf_ref[...])
    r = xn.shape[0]
    s_dp = jax.lax.dot_general(xn, mu_dp, (((1,), (1,)), ((), ())),
                               preferred_element_type=jnp.float32)
    s_cf = jax.lax.dot_general(yn, mu_cf, (((1,), (1,)), ((), ())),
                               preferred_element_type=jnp.float32)

    lab = lab_ref[0, 0, :]
    onehot = (lab[:, None] == jax.lax.broadcasted_iota(jnp.int32, (r, NSEG), 1)
              ).astype(jnp.float32)
    s_lab_dp = jnp.sum(s_dp * onehot, axis=1, keepdims=True)
    s_lab_cf = jnp.sum(s_cf * onehot, axis=1, keepdims=True)
    s_bg_dp = s_dp[:, 0:1]
    s_bg_cf = s_cf[:, 0:1]

    per_t = jnp.log1p(jnp.exp((s_bg_dp - s_lab_dp) / TAU))
    per_c = jnp.log1p(jnp.exp((s_lab_cf - s_bg_cf) / TAU))

    t_contrib = jnp.sum(onehot * per_t, axis=0)
    c_contrib = jnp.sum(onehot * per_c, axis=0)

    @pl.when(step == 0)
    def _():
        tsum_ref[...] = jnp.zeros_like(tsum_ref)
        csum_ref[...] = jnp.zeros_like(csum_ref)

    tsum_ref[...] += jnp.broadcast_to(t_contrib[:, None], tsum_ref.shape)
    csum_ref[...] += jnp.broadcast_to(c_contrib[:, None], csum_ref.shape)


def kernel(dp, f_cf, patch_mask):
    n, d = dp.shape
    r = _ROWS
    assert n % r == 0
    nb = n // r
    lab3 = patch_mask.reshape(nb, 1, r)

    row_spec = pl.BlockSpec((r, d), lambda i: (i, 0))
    lab_spec = pl.BlockSpec((1, 1, r), lambda i: (i, 0, 0))
    acc_spec = pl.BlockSpec((NSEG, d), lambda i: (0, 0))

    segdp, segcf, cnt = pl.pallas_call(
        _pass1,
        grid=(nb,),
        in_specs=[row_spec, row_spec, lab_spec],
        out_specs=[acc_spec, acc_spec, acc_spec],
        out_shape=[jax.ShapeDtypeStruct((NSEG, d), jnp.float32)] * 3,
    )(dp, f_cf, lab3)

    tsum, csum = pl.pallas_call(
        _pass2,
        grid=(nb,),
        in_specs=[row_spec, row_spec, lab_spec, acc_spec, acc_spec, acc_spec],
        out_specs=[acc_spec, acc_spec],
        out_shape=[jax.ShapeDtypeStruct((NSEG, d), jnp.float32)] * 2,
    )(dp, f_cf, lab3, segdp, segcf, cnt)

    counts = cnt[1:NUM_INST + 1, 0]
    valid = (counts >= MIN_PIXELS).astype(jnp.float32)
    safe = jnp.maximum(counts, 1.0)
    loss_t = jnp.sum(valid * tsum[1:NUM_INST + 1, 0] / safe) / jnp.sum(valid)
    loss_c = jnp.sum(valid * csum[1:NUM_INST + 1, 0] / safe) / jnp.sum(valid)
    return loss_t + LAMBDA_CF * loss_c


# speedup vs baseline: 11.2257x; 2.1122x over previous
import jax
import jax.numpy as jnp
from jax.experimental import pallas as pl
from jax.experimental.pallas import tpu as pltpu

TAU = 0.07
MIN_PIXELS = 3
LAMBDA_CF = 0.5
NUM_INST = 8
NSEG = 16

_ROWS = 4000


def _inv_norm_t(x):
    xsq = x * x
    ones8 = jnp.ones((8, 128), jnp.float32)
    ss_t = jax.lax.dot_general(ones8, xsq, (((1,), (1,)), ((), ())),
                               preferred_element_type=jnp.float32)
    return jax.lax.rsqrt(jnp.maximum(ss_t[0:1], 1e-24))


def _onehot_t(lab, r):
    iot = jax.lax.broadcasted_iota(jnp.int32, (NSEG, r), 0)
    return (jnp.broadcast_to(lab, (NSEG, r)) == iot).astype(jnp.float32)


def _pass1(dp_ref, cf_ref, lab_ref, segdp_ref, segcf_ref, cnt_ref):
    step = pl.program_id(0)
    x = dp_ref[...]
    y = cf_ref[...]
    lab = lab_ref[0]
    r = x.shape[0]
    oh = _onehot_t(lab, r)
    wd = oh * _inv_norm_t(x)
    wc = oh * _inv_norm_t(y)
    sdp = jax.lax.dot_general(wd, x, (((1,), (0,)), ((), ())),
                              preferred_element_type=jnp.float32)
    scf = jax.lax.dot_general(wc, y, (((1,), (0,)), ((), ())),
                              preferred_element_type=jnp.float32)
    cnt = jnp.sum(oh, axis=1, keepdims=True)

    @pl.when(step == 0)
    def _():
        segdp_ref[...] = jnp.zeros_like(segdp_ref)
        segcf_ref[...] = jnp.zeros_like(segcf_ref)
        cnt_ref[...] = jnp.zeros_like(cnt_ref)

    segdp_ref[...] += sdp
    segcf_ref[...] += scf
    cnt_ref[...] += jnp.broadcast_to(cnt, cnt_ref.shape)


def _mu(seg, safe):
    m = seg / safe
    n = jnp.sqrt(jnp.sum(m * m, axis=1, keepdims=True))
    return m / jnp.maximum(n, 1e-12)


def _pass2(dp_ref, cf_ref, lab_ref, segdp_ref, segcf_ref, cnt_ref,
           tsum_ref, csum_ref):
    step = pl.program_id(0)
    counts = cnt_ref[:, 0:1]
    safe = jnp.maximum(counts, 1.0)
    mu_dp = _mu(segdp_ref[...], safe)
    mu_cf = _mu(segcf_ref[...], safe)

    x = dp_ref[...]
    y = cf_ref[...]
    r = x.shape[0]
    lab = lab_ref[0]
    oh = _onehot_t(lab, r)

    st_d = jax.lax.dot_general(mu_dp, x, (((1,), (1,)), ((), ())),
                               preferred_element_type=jnp.float32)
    st_c = jax.lax.dot_general(mu_cf, y, (((1,), (1,)), ((), ())),
                               preferred_element_type=jnp.float32)
    st_d = st_d * _inv_norm_t(x)
    st_c = st_c * _inv_norm_t(y)

    s_lab_d = jnp.sum(st_d * oh, axis=0, keepdims=True)
    s_lab_c = jnp.sum(st_c * oh, axis=0, keepdims=True)
    z_d = (st_d[0:1] - s_lab_d) * (1.0 / TAU)
    z_c = (s_lab_c - st_c[0:1]) * (1.0 / TAU)
    per_t = jnp.log1p(jnp.exp(z_d))
    per_c = jnp.log1p(jnp.exp(z_c))

    t_contrib = jnp.sum(oh * per_t, axis=1, keepdims=True)
    c_contrib = jnp.sum(oh * per_c, axis=1, keepdims=True)

    @pl.when(step == 0)
    def _():
        tsum_ref[...] = jnp.zeros_like(tsum_ref)
        csum_ref[...] = jnp.zeros_like(csum_ref)

    tsum_ref[...] += jnp.broadcast_to(t_contrib, tsum_ref.shape)
    csum_ref[...] += jnp.broadcast_to(c_contrib, csum_ref.shape)


def kernel(dp, f_cf, patch_mask):
    n, d = dp.shape
    r = _ROWS
    assert n % r == 0
    nb = n // r
    lab3 = patch_mask.reshape(nb, 1, r)

    row_spec = pl.BlockSpec((r, d), lambda i: (i, 0))
    lab_spec = pl.BlockSpec((1, 1, r), lambda i: (i, 0, 0))
    acc_spec = pl.BlockSpec((NSEG, d), lambda i: (0, 0))

    segdp, segcf, cnt = pl.pallas_call(
        _pass1,
        grid=(nb,),
        in_specs=[row_spec, row_spec, lab_spec],
        out_specs=[acc_spec, acc_spec, acc_spec],
        out_shape=[jax.ShapeDtypeStruct((NSEG, d), jnp.float32)] * 3,
    )(dp, f_cf, lab3)

    tsum, csum = pl.pallas_call(
        _pass2,
        grid=(nb,),
        in_specs=[row_spec, row_spec, lab_spec, acc_spec, acc_spec, acc_spec],
        out_specs=[acc_spec, acc_spec],
        out_shape=[jax.ShapeDtypeStruct((NSEG, d), jnp.float32)] * 2,
    )(dp, f_cf, lab3, segdp, segcf, cnt)

    counts = cnt[1:NUM_INST + 1, 0]
    valid = (counts >= MIN_PIXELS).astype(jnp.float32)
    safe = jnp.maximum(counts, 1.0)
    loss_t = jnp.sum(valid * tsum[1:NUM_INST + 1, 0] / safe) / jnp.sum(valid)
    loss_c = jnp.sum(valid * csum[1:NUM_INST + 1, 0] / safe) / jnp.sum(valid)
    return loss_t + LAMBDA_CF * loss_c


# speedup vs baseline: 13.5290x; 1.2052x over previous
import jax
import jax.numpy as jnp
from jax.experimental import pallas as pl
from jax.experimental.pallas import tpu as pltpu

TAU = 0.07
MIN_PIXELS = 3
LAMBDA_CF = 0.5
NUM_INST = 8
NSEG = 16

_ROWS = 8000


def _inv_norm_t(x):
    xsq = x * x
    ones8 = jnp.ones((8, 128), jnp.float32)
    ss_t = jax.lax.dot_general(ones8, xsq, (((1,), (1,)), ((), ())),
                               preferred_element_type=jnp.float32)
    return jax.lax.rsqrt(jnp.maximum(ss_t[0:1], 1e-24))


def _onehot_t(lab, r):
    iot = jax.lax.broadcasted_iota(jnp.int32, (NSEG, r), 0)
    return (jnp.broadcast_to(lab, (NSEG, r)) == iot).astype(jnp.float32)


def _pass1(dp_ref, cf_ref, lab_ref, segdp_ref, segcf_ref, cnt_ref):
    step = pl.program_id(0)
    x = dp_ref[...]
    y = cf_ref[...]
    lab = lab_ref[0]
    r = x.shape[0]
    oh = _onehot_t(lab, r)
    wd = oh * _inv_norm_t(x)
    wc = oh * _inv_norm_t(y)
    sdp = jax.lax.dot_general(wd, x, (((1,), (0,)), ((), ())),
                              preferred_element_type=jnp.float32)
    scf = jax.lax.dot_general(wc, y, (((1,), (0,)), ((), ())),
                              preferred_element_type=jnp.float32)
    cnt = jnp.sum(oh, axis=1, keepdims=True)

    @pl.when(step == 0)
    def _():
        segdp_ref[...] = jnp.zeros_like(segdp_ref)
        segcf_ref[...] = jnp.zeros_like(segcf_ref)
        cnt_ref[...] = jnp.zeros_like(cnt_ref)

    segdp_ref[...] += sdp
    segcf_ref[...] += scf
    cnt_ref[...] += jnp.broadcast_to(cnt, cnt_ref.shape)


def _mu(seg, safe):
    m = seg / safe
    n = jnp.sqrt(jnp.sum(m * m, axis=1, keepdims=True))
    return m / jnp.maximum(n, 1e-12)


def _pass2(dp_ref, cf_ref, lab_ref, segdp_ref, segcf_ref, cnt_ref,
           tsum_ref, csum_ref):
    step = pl.program_id(0)
    counts = cnt_ref[:, 0:1]
    safe = jnp.maximum(counts, 1.0)
    mu_dp = _mu(segdp_ref[...], safe)
    mu_cf = _mu(segcf_ref[...], safe)

    x = dp_ref[...]
    y = cf_ref[...]
    r = x.shape[0]
    lab = lab_ref[0]
    oh = _onehot_t(lab, r)

    st_d = jax.lax.dot_general(mu_dp, x, (((1,), (1,)), ((), ())),
                               preferred_element_type=jnp.float32)
    st_c = jax.lax.dot_general(mu_cf, y, (((1,), (1,)), ((), ())),
                               preferred_element_type=jnp.float32)
    st_d = st_d * _inv_norm_t(x)
    st_c = st_c * _inv_norm_t(y)

    s_lab_d = jnp.sum(st_d * oh, axis=0, keepdims=True)
    s_lab_c = jnp.sum(st_c * oh, axis=0, keepdims=True)
    z_d = (st_d[0:1] - s_lab_d) * (1.0 / TAU)
    z_c = (s_lab_c - st_c[0:1]) * (1.0 / TAU)
    per_t = jnp.log1p(jnp.exp(z_d))
    per_c = jnp.log1p(jnp.exp(z_c))

    t_contrib = jnp.sum(oh * per_t, axis=1, keepdims=True)
    c_contrib = jnp.sum(oh * per_c, axis=1, keepdims=True)

    @pl.when(step == 0)
    def _():
        tsum_ref[...] = jnp.zeros_like(tsum_ref)
        csum_ref[...] = jnp.zeros_like(csum_ref)

    tsum_ref[...] += jnp.broadcast_to(t_contrib, tsum_ref.shape)
    csum_ref[...] += jnp.broadcast_to(c_contrib, csum_ref.shape)


def kernel(dp, f_cf, patch_mask):
    n, d = dp.shape
    r = _ROWS
    assert n % r == 0
    nb = n // r
    lab3 = patch_mask.reshape(nb, 1, r)

    row_spec = pl.BlockSpec((r, d), lambda i: (i, 0))
    lab_spec = pl.BlockSpec((1, 1, r), lambda i: (i, 0, 0))
    acc_spec = pl.BlockSpec((NSEG, d), lambda i: (0, 0))

    segdp, segcf, cnt = pl.pallas_call(
        _pass1,
        grid=(nb,),
        in_specs=[row_spec, row_spec, lab_spec],
        out_specs=[acc_spec, acc_spec, acc_spec],
        out_shape=[jax.ShapeDtypeStruct((NSEG, d), jnp.float32)] * 3,
    )(dp, f_cf, lab3)

    tsum, csum = pl.pallas_call(
        _pass2,
        grid=(nb,),
        in_specs=[row_spec, row_spec, lab_spec, acc_spec, acc_spec, acc_spec],
        out_specs=[acc_spec, acc_spec],
        out_shape=[jax.ShapeDtypeStruct((NSEG, d), jnp.float32)] * 2,
    )(dp, f_cf, lab3, segdp, segcf, cnt)

    counts = cnt[1:NUM_INST + 1, 0]
    valid = (counts >= MIN_PIXELS).astype(jnp.float32)
    safe = jnp.maximum(counts, 1.0)
    loss_t = jnp.sum(valid * tsum[1:NUM_INST + 1, 0] / safe) / jnp.sum(valid)
    loss_c = jnp.sum(valid * csum[1:NUM_INST + 1, 0] / safe) / jnp.sum(valid)
    return loss_t + LAMBDA_CF * loss_c


# speedup vs baseline: 14.7231x; 1.0883x over previous
import jax
import jax.numpy as jnp
from jax.experimental import pallas as pl
from jax.experimental.pallas import tpu as pltpu

TAU = 0.07
MIN_PIXELS = 3
LAMBDA_CF = 0.5
NUM_INST = 8
NSEG = 16

_ROWS = 16000


def _inv_norm_t(x):
    xsq = x * x
    ones8 = jnp.ones((8, 128), jnp.float32)
    ss_t = jax.lax.dot_general(ones8, xsq, (((1,), (1,)), ((), ())),
                               preferred_element_type=jnp.float32)
    return jax.lax.rsqrt(jnp.maximum(ss_t[0:1], 1e-24))


def _onehot_t(lab, r):
    iot = jax.lax.broadcasted_iota(jnp.int32, (NSEG, r), 0)
    return (jnp.broadcast_to(lab, (NSEG, r)) == iot).astype(jnp.float32)


def _pass1(dp_ref, cf_ref, lab_ref, segdp_ref, segcf_ref, cnt_ref):
    step = pl.program_id(0)
    x = dp_ref[...]
    y = cf_ref[...]
    lab = lab_ref[0]
    r = x.shape[0]
    oh = _onehot_t(lab, r)
    wd = oh * _inv_norm_t(x)
    wc = oh * _inv_norm_t(y)
    sdp = jax.lax.dot_general(wd, x, (((1,), (0,)), ((), ())),
                              preferred_element_type=jnp.float32)
    scf = jax.lax.dot_general(wc, y, (((1,), (0,)), ((), ())),
                              preferred_element_type=jnp.float32)
    cnt = jnp.sum(oh, axis=1, keepdims=True)

    @pl.when(step == 0)
    def _():
        segdp_ref[...] = jnp.zeros_like(segdp_ref)
        segcf_ref[...] = jnp.zeros_like(segcf_ref)
        cnt_ref[...] = jnp.zeros_like(cnt_ref)

    segdp_ref[...] += sdp
    segcf_ref[...] += scf
    cnt_ref[...] += jnp.broadcast_to(cnt, cnt_ref.shape)


def _mu(seg, safe):
    m = seg / safe
    n = jnp.sqrt(jnp.sum(m * m, axis=1, keepdims=True))
    return m / jnp.maximum(n, 1e-12)


def _pass2(dp_ref, cf_ref, lab_ref, segdp_ref, segcf_ref, cnt_ref,
           tsum_ref, csum_ref):
    step = pl.program_id(0)
    counts = cnt_ref[:, 0:1]
    safe = jnp.maximum(counts, 1.0)
    mu_dp = _mu(segdp_ref[...], safe)
    mu_cf = _mu(segcf_ref[...], safe)

    x = dp_ref[...]
    y = cf_ref[...]
    r = x.shape[0]
    lab = lab_ref[0]
    oh = _onehot_t(lab, r)

    st_d = jax.lax.dot_general(mu_dp, x, (((1,), (1,)), ((), ())),
                               preferred_element_type=jnp.float32)
    st_c = jax.lax.dot_general(mu_cf, y, (((1,), (1,)), ((), ())),
                               preferred_element_type=jnp.float32)
    st_d = st_d * _inv_norm_t(x)
    st_c = st_c * _inv_norm_t(y)

    s_lab_d = jnp.sum(st_d * oh, axis=0, keepdims=True)
    s_lab_c = jnp.sum(st_c * oh, axis=0, keepdims=True)
    z_d = (st_d[0:1] - s_lab_d) * (1.0 / TAU)
    z_c = (s_lab_c - st_c[0:1]) * (1.0 / TAU)
    per_t = jnp.log1p(jnp.exp(z_d))
    per_c = jnp.log1p(jnp.exp(z_c))

    t_contrib = jnp.sum(oh * per_t, axis=1, keepdims=True)
    c_contrib = jnp.sum(oh * per_c, axis=1, keepdims=True)

    @pl.when(step == 0)
    def _():
        tsum_ref[...] = jnp.zeros_like(tsum_ref)
        csum_ref[...] = jnp.zeros_like(csum_ref)

    tsum_ref[...] += jnp.broadcast_to(t_contrib, tsum_ref.shape)
    csum_ref[...] += jnp.broadcast_to(c_contrib, csum_ref.shape)


def kernel(dp, f_cf, patch_mask):
    n, d = dp.shape
    r = _ROWS
    assert n % r == 0
    nb = n // r
    lab3 = patch_mask.reshape(nb, 1, r)

    row_spec = pl.BlockSpec((r, d), lambda i: (i, 0))
    lab_spec = pl.BlockSpec((1, 1, r), lambda i: (i, 0, 0))
    acc_spec = pl.BlockSpec((NSEG, d), lambda i: (0, 0))

    segdp, segcf, cnt = pl.pallas_call(
        _pass1,
        grid=(nb,),
        in_specs=[row_spec, row_spec, lab_spec],
        out_specs=[acc_spec, acc_spec, acc_spec],
        out_shape=[jax.ShapeDtypeStruct((NSEG, d), jnp.float32)] * 3,
    )(dp, f_cf, lab3)

    tsum, csum = pl.pallas_call(
        _pass2,
        grid=(nb,),
        in_specs=[row_spec, row_spec, lab_spec, acc_spec, acc_spec, acc_spec],
        out_specs=[acc_spec, acc_spec],
        out_shape=[jax.ShapeDtypeStruct((NSEG, d), jnp.float32)] * 2,
    )(dp, f_cf, lab3, segdp, segcf, cnt)

    counts = cnt[1:NUM_INST + 1, 0]
    valid = (counts >= MIN_PIXELS).astype(jnp.float32)
    safe = jnp.maximum(counts, 1.0)
    loss_t = jnp.sum(valid * tsum[1:NUM_INST + 1, 0] / safe) / jnp.sum(valid)
    loss_c = jnp.sum(valid * csum[1:NUM_INST + 1, 0] / safe) / jnp.sum(valid)
    return loss_t + LAMBDA_CF * loss_c


# speedup vs baseline: 14.7527x; 1.0020x over previous
import jax
import jax.numpy as jnp
from jax.experimental import pallas as pl
from jax.experimental.pallas import tpu as pltpu

TAU = 0.07
MIN_PIXELS = 3
LAMBDA_CF = 0.5
NUM_INST = 8
NSEG = 16

_ROWS = 20000


def _inv_norm_t(x):
    xsq = x * x
    ones8 = jnp.ones((8, 128), jnp.float32)
    ss_t = jax.lax.dot_general(ones8, xsq, (((1,), (1,)), ((), ())),
                               preferred_element_type=jnp.float32)
    return jax.lax.rsqrt(jnp.maximum(ss_t[0:1], 1e-24))


def _onehot_t(lab, r):
    iot = jax.lax.broadcasted_iota(jnp.int32, (NSEG, r), 0)
    return (jnp.broadcast_to(lab, (NSEG, r)) == iot).astype(jnp.float32)


def _pass1(dp_ref, cf_ref, lab_ref, segdp_ref, segcf_ref, cnt_ref):
    step = pl.program_id(0)
    x = dp_ref[...]
    y = cf_ref[...]
    lab = lab_ref[0]
    r = x.shape[0]
    oh = _onehot_t(lab, r)
    wd = oh * _inv_norm_t(x)
    wc = oh * _inv_norm_t(y)
    sdp = jax.lax.dot_general(wd, x, (((1,), (0,)), ((), ())),
                              preferred_element_type=jnp.float32)
    scf = jax.lax.dot_general(wc, y, (((1,), (0,)), ((), ())),
                              preferred_element_type=jnp.float32)
    cnt = jnp.sum(oh, axis=1, keepdims=True)

    @pl.when(step == 0)
    def _():
        segdp_ref[...] = jnp.zeros_like(segdp_ref)
        segcf_ref[...] = jnp.zeros_like(segcf_ref)
        cnt_ref[...] = jnp.zeros_like(cnt_ref)

    segdp_ref[...] += sdp
    segcf_ref[...] += scf
    cnt_ref[...] += jnp.broadcast_to(cnt, cnt_ref.shape)


def _mu(seg, safe):
    m = seg / safe
    n = jnp.sqrt(jnp.sum(m * m, axis=1, keepdims=True))
    return m / jnp.maximum(n, 1e-12)


def _pass2(dp_ref, cf_ref, lab_ref, segdp_ref, segcf_ref, cnt_ref,
           tsum_ref, csum_ref):
    step = pl.program_id(0)
    counts = cnt_ref[:, 0:1]
    safe = jnp.maximum(counts, 1.0)
    mu_dp = _mu(segdp_ref[...], safe)
    mu_cf = _mu(segcf_ref[...], safe)

    x = dp_ref[...]
    y = cf_ref[...]
    r = x.shape[0]
    lab = lab_ref[0]
    oh = _onehot_t(lab, r)

    st_d = jax.lax.dot_general(mu_dp, x, (((1,), (1,)), ((), ())),
                               preferred_element_type=jnp.float32)
    st_c = jax.lax.dot_general(mu_cf, y, (((1,), (1,)), ((), ())),
                               preferred_element_type=jnp.float32)
    st_d = st_d * _inv_norm_t(x)
    st_c = st_c * _inv_norm_t(y)

    s_lab_d = jnp.sum(st_d * oh, axis=0, keepdims=True)
    s_lab_c = jnp.sum(st_c * oh, axis=0, keepdims=True)
    z_d = (st_d[0:1] - s_lab_d) * (1.0 / TAU)
    z_c = (s_lab_c - st_c[0:1]) * (1.0 / TAU)
    per_t = jnp.log1p(jnp.exp(z_d))
    per_c = jnp.log1p(jnp.exp(z_c))

    t_contrib = jnp.sum(oh * per_t, axis=1, keepdims=True)
    c_contrib = jnp.sum(oh * per_c, axis=1, keepdims=True)

    @pl.when(step == 0)
    def _():
        tsum_ref[...] = jnp.zeros_like(tsum_ref)
        csum_ref[...] = jnp.zeros_like(csum_ref)

    tsum_ref[...] += jnp.broadcast_to(t_contrib, tsum_ref.shape)
    csum_ref[...] += jnp.broadcast_to(c_contrib, csum_ref.shape)


def kernel(dp, f_cf, patch_mask):
    n, d = dp.shape
    r = _ROWS
    assert n % r == 0
    nb = n // r
    lab3 = patch_mask.reshape(nb, 1, r)

    row_spec = pl.BlockSpec((r, d), lambda i: (i, 0))
    lab_spec = pl.BlockSpec((1, 1, r), lambda i: (i, 0, 0))
    acc_spec = pl.BlockSpec((NSEG, d), lambda i: (0, 0))

    segdp, segcf, cnt = pl.pallas_call(
        _pass1,
        grid=(nb,),
        in_specs=[row_spec, row_spec, lab_spec],
        out_specs=[acc_spec, acc_spec, acc_spec],
        out_shape=[jax.ShapeDtypeStruct((NSEG, d), jnp.float32)] * 3,
    )(dp, f_cf, lab3)

    tsum, csum = pl.pallas_call(
        _pass2,
        grid=(nb,),
        in_specs=[row_spec, row_spec, lab_spec, acc_spec, acc_spec, acc_spec],
        out_specs=[acc_spec, acc_spec],
        out_shape=[jax.ShapeDtypeStruct((NSEG, d), jnp.float32)] * 2,
    )(dp, f_cf, lab3, segdp, segcf, cnt)

    counts = cnt[1:NUM_INST + 1, 0]
    valid = (counts >= MIN_PIXELS).astype(jnp.float32)
    safe = jnp.maximum(counts, 1.0)
    loss_t = jnp.sum(valid * tsum[1:NUM_INST + 1, 0] / safe) / jnp.sum(valid)
    loss_c = jnp.sum(valid * csum[1:NUM_INST + 1, 0] / safe) / jnp.sum(valid)
    return loss_t + LAMBDA_CF * loss_c
